# baseline (speedup 1.0000x reference)
import numpy as np

import jax
import jax.numpy as jnp
from jax import lax
from jax.experimental import pallas as pl
from jax.experimental.pallas import tpu as pltpu
from jax.experimental.pallas import tpu_sc as plsc

BS, NM, TS, D = 16384, 20, 8, 3
ROWS = NM * D * TS
B1 = 4096
_NC, _NS = 2, 16
_NW = _NC * _NS
_CH = 128
SC_LANES = 12288
_NCH = SC_LANES // (_NW * _CH)
TC_BLK0 = SC_LANES // B1


def _pair_mask():
    m = np.zeros((NM, NM * 16), np.float32)
    for mm in range(NM):
        for t in range(TS):
            m[mm, mm * 16 + 2 * t] = 1.0
    return m


_M_NP = _pair_mask()


def _k1_body(pa_ref, tg_ref, cls_ref, m_ref, mi_ref, accc_ref):
    @pl.when(pl.program_id(0) == 0)
    def _():
        accc_ref[...] = jnp.zeros_like(accc_ref)

    b = pa_ref.shape[2]
    tx = tg_ref[pl.ds(0, TS), :]
    ty = tg_ref[pl.ds(TS, TS), :]
    tgi = jnp.concatenate(
        [tx[:, None, :], ty[:, None, :]], axis=1).reshape(16, b)
    pa = pa_ref[...].reshape(NM, 16, b)
    d = pa - tgi[None]
    d2 = (d * d).reshape(NM * 16, b)
    sh = jnp.concatenate([d2[1:], d2[:1]], axis=0)
    h = jnp.sqrt(d2 + sh)
    dist = jnp.dot(m_ref[...], h, preferred_element_type=jnp.float32)
    mn = jnp.min(dist, axis=0, keepdims=True)
    i20 = lax.broadcasted_iota(jnp.int32, dist.shape, 0)
    cand = jnp.where(dist == mn, i20, jnp.int32(1 << 30))
    mi = jnp.min(cand, axis=0, keepdims=True)
    mi_ref[...] = jnp.broadcast_to(mi, (8, b))

    x = cls_ref[...]
    t = (i20 == mi).astype(jnp.float32)
    p = jax.nn.sigmoid(x)
    pt = (1.0 - p) * t + p * (1.0 - t)
    w = (0.25 * t + 0.75 * (1.0 - t)) * pt * pt
    bce = jnp.maximum(x, 0.0) - x * t + jnp.log1p(jnp.exp(-jnp.abs(x)))
    accc_ref[...] += jnp.sum(bce * w).reshape(1, 1)


def _k3b_body(pr_ref, tg_ref, mi_ref, accl_ref):
    @pl.when(pl.program_id(0) == 0)
    def _():
        accl_ref[...] = jnp.zeros_like(accl_ref)

    tg = tg_ref[...]
    mi = mi_ref[pl.ds(0, 1), :]
    acc = jnp.zeros(mi.shape, jnp.float32)
    for m in range(NM):
        d = jnp.abs(pr_ref[pl.ds(m * 24, 24), :] - tg)
        s = jnp.sum(d, axis=0, keepdims=True)
        acc = acc + jnp.where(mi == m, s, 0.0)
    accl_ref[...] += jnp.sum(acc).reshape(1, 1)


def _k2_body(pr_hbm, tg_hbm, mi_hbm, out_hbm, slabs, tgv, miv, accv, sems):
    wid = lax.axis_index("s") * _NC + lax.axis_index("c")
    col16 = lax.iota(jnp.int32, 16)
    base = wid * _NCH * _CH

    def start_slab(chunk, buf):
        c0 = base + chunk * _CH
        return pltpu.async_copy(
            pr_hbm.at[:, pl.ds(c0, _CH)], slabs.at[buf], sems.at[buf])

    acc = jnp.zeros((16,), jnp.float32)
    pend = start_slab(0, 0)
    for chunk in range(_NCH):
        buf = chunk % 2
        if chunk + 1 < _NCH:
            nxt = start_slab(chunk + 1, 1 - buf)
        c0 = base + chunk * _CH
        pltpu.sync_copy(tg_hbm.at[:, pl.ds(c0, _CH)], tgv)
        pltpu.sync_copy(mi_hbm.at[:, pl.ds(c0, _CH)], miv)
        pend.wait()
        if chunk + 1 < _NCH:
            pend = nxt

        def body(g, s):
            mi16 = miv[0, pl.ds(g * 16, 16)]
            rowb = mi16 * (TS * D)
            cols = col16 + g * 16
            for r in range(TS * D):
                v = plsc.load_gather(slabs.at[buf], [rowb + r, cols])
                s = s + jnp.abs(v - tgv[r, pl.ds(g * 16, 16)])
            return s

        acc = lax.fori_loop(0, _CH // 16, body, acc)
    accv[...] = acc
    pltpu.sync_copy(accv, out_hbm.at[wid])


def _make_k2():
    mesh = plsc.VectorSubcoreMesh(core_axis_name="c", subcore_axis_name="s")
    return pl.kernel(
        _k2_body,
        out_type=jax.ShapeDtypeStruct((_NW, 16), jnp.float32),
        mesh=mesh,
        scratch_types=[
            pltpu.VMEM((2, ROWS, _CH), jnp.float32),
            pltpu.VMEM((D * TS, _CH), jnp.float32),
            pltpu.VMEM((8, _CH), jnp.int32),
            pltpu.VMEM((16,), jnp.float32),
            pltpu.SemaphoreType.DMA((2,)),
        ],
        compiler_params=pltpu.CompilerParams(needs_layout_passes=False),
    )


def kernel(poses_reg, poses_cls, targets, plan_anchor):
    pa_t = jnp.transpose(plan_anchor, (1, 2, 3, 0)).reshape(NM * TS, 2, BS)
    pr_t = jnp.transpose(poses_reg, (1, 3, 2, 0)).reshape(ROWS, BS)
    tg_t = jnp.transpose(targets, (2, 1, 0)).reshape(D * TS, BS)
    cls_t = jnp.transpose(poses_cls, (1, 0))
    m_c = jnp.asarray(_M_NP)

    mi8, accc = pl.pallas_call(
        _k1_body,
        grid=(BS // B1,),
        in_specs=[
            pl.BlockSpec((NM * TS, 2, B1), lambda i: (0, 0, i)),
            pl.BlockSpec((D * TS, B1), lambda i: (0, i)),
            pl.BlockSpec((NM, B1), lambda i: (0, i)),
            pl.BlockSpec((NM, NM * 16), lambda i: (0, 0)),
        ],
        out_specs=[
            pl.BlockSpec((8, B1), lambda i: (0, i)),
            pl.BlockSpec((1, 1), lambda i: (0, 0)),
        ],
        out_shape=[
            jax.ShapeDtypeStruct((8, BS), jnp.int32),
            jax.ShapeDtypeStruct((1, 1), jnp.float32),
        ],
    )(pa_t, tg_t, cls_t, m_c)

    l1p = _make_k2()(pr_t, tg_t, mi8)

    if SC_LANES < BS:
        accl = pl.pallas_call(
            _k3b_body,
            grid=((BS - SC_LANES) // B1,),
            in_specs=[
                pl.BlockSpec((ROWS, B1), lambda i: (0, i + TC_BLK0)),
                pl.BlockSpec((D * TS, B1), lambda i: (0, i + TC_BLK0)),
                pl.BlockSpec((8, B1), lambda i: (0, i + TC_BLK0)),
            ],
            out_specs=pl.BlockSpec((1, 1), lambda i: (0, 0)),
            out_shape=jax.ShapeDtypeStruct((1, 1), jnp.float32),
        )(pr_t, tg_t, mi8)
        l1_tc = accl[0, 0]
    else:
        l1_tc = jnp.float32(0.0)

    return (accc[0, 0] * (10.0 / (BS * NM))
            + (l1_tc + jnp.sum(l1p)) * (1.0 / (BS * TS * D)))


# speedup vs baseline: 1.0034x; 1.0034x over previous
import numpy as np

import jax
import jax.numpy as jnp
from jax import lax
from jax.experimental import pallas as pl
from jax.experimental.pallas import tpu as pltpu
from jax.experimental.pallas import tpu_sc as plsc

BS, NM, TS, D = 16384, 20, 8, 3
ROWS = NM * D * TS
B1 = 2048
_NC, _NS = 2, 16
_NW = _NC * _NS
_CH = 128
SC_LANES = 12288
_NCH = SC_LANES // (_NW * _CH)
TC_BLK0 = SC_LANES // B1


def _pair_mask():
    m = np.zeros((NM, NM * 16), np.float32)
    for mm in range(NM):
        for t in range(TS):
            m[mm, mm * 16 + 2 * t] = 1.0
    return m


_M_NP = _pair_mask()


def _k1_body(pa_ref, tg_ref, cls_ref, m_ref, mi_ref, accc_ref):
    @pl.when(pl.program_id(0) == 0)
    def _():
        accc_ref[...] = jnp.zeros_like(accc_ref)

    b = pa_ref.shape[2]
    tx = tg_ref[pl.ds(0, TS), :]
    ty = tg_ref[pl.ds(TS, TS), :]
    tgi = jnp.concatenate(
        [tx[:, None, :], ty[:, None, :]], axis=1).reshape(16, b)
    pa = pa_ref[...].reshape(NM, 16, b)
    d = pa - tgi[None]
    d2 = (d * d).reshape(NM * 16, b)
    sh = jnp.concatenate([d2[1:], d2[:1]], axis=0)
    h = jnp.sqrt(d2 + sh)
    dist = jnp.dot(m_ref[...], h, preferred_element_type=jnp.float32)
    mn = jnp.min(dist, axis=0, keepdims=True)
    i20 = lax.broadcasted_iota(jnp.int32, dist.shape, 0)
    cand = jnp.where(dist == mn, i20, jnp.int32(1 << 30))
    mi = jnp.min(cand, axis=0, keepdims=True)
    mi_ref[...] = jnp.broadcast_to(mi, (8, b))

    x = cls_ref[...]
    t = (i20 == mi).astype(jnp.float32)
    p = jax.nn.sigmoid(x)
    pt = (1.0 - p) * t + p * (1.0 - t)
    w = (0.25 * t + 0.75 * (1.0 - t)) * pt * pt
    bce = jnp.maximum(x, 0.0) - x * t + jnp.log1p(jnp.exp(-jnp.abs(x)))
    accc_ref[...] += jnp.sum(bce * w).reshape(1, 1)


def _k3b_body(pr_ref, tg_ref, mi_ref, accl_ref):
    @pl.when(pl.program_id(0) == 0)
    def _():
        accl_ref[...] = jnp.zeros_like(accl_ref)

    tg = tg_ref[...]
    mi = mi_ref[pl.ds(0, 1), :]
    acc = jnp.zeros(mi.shape, jnp.float32)
    for m in range(NM):
        d = jnp.abs(pr_ref[pl.ds(m * 24, 24), :] - tg)
        s = jnp.sum(d, axis=0, keepdims=True)
        acc = acc + jnp.where(mi == m, s, 0.0)
    accl_ref[...] += jnp.sum(acc).reshape(1, 1)


def _k2_body(pr_hbm, tg_hbm, mi_hbm, out_hbm, slabs, tgv, miv, accv, sems):
    wid = lax.axis_index("s") * _NC + lax.axis_index("c")
    col16 = lax.iota(jnp.int32, 16)
    base = wid * _NCH * _CH

    def start_slab(chunk, buf):
        c0 = base + chunk * _CH
        return pltpu.async_copy(
            pr_hbm.at[:, pl.ds(c0, _CH)], slabs.at[buf], sems.at[buf])

    acc = jnp.zeros((16,), jnp.float32)
    pend = start_slab(0, 0)
    for chunk in range(_NCH):
        buf = chunk % 2
        if chunk + 1 < _NCH:
            nxt = start_slab(chunk + 1, 1 - buf)
        c0 = base + chunk * _CH
        pltpu.sync_copy(tg_hbm.at[:, pl.ds(c0, _CH)], tgv)
        pltpu.sync_copy(mi_hbm.at[:, pl.ds(c0, _CH)], miv)
        pend.wait()
        if chunk + 1 < _NCH:
            pend = nxt

        def body(g, s):
            mi16 = miv[0, pl.ds(g * 16, 16)]
            rowb = mi16 * (TS * D)
            cols = col16 + g * 16
            for r in range(TS * D):
                v = plsc.load_gather(slabs.at[buf], [rowb + r, cols])
                s = s + jnp.abs(v - tgv[r, pl.ds(g * 16, 16)])
            return s

        acc = lax.fori_loop(0, _CH // 16, body, acc)
    accv[...] = acc
    pltpu.sync_copy(accv, out_hbm.at[wid])


def _make_k2():
    mesh = plsc.VectorSubcoreMesh(core_axis_name="c", subcore_axis_name="s")
    return pl.kernel(
        _k2_body,
        out_type=jax.ShapeDtypeStruct((_NW, 16), jnp.float32),
        mesh=mesh,
        scratch_types=[
            pltpu.VMEM((2, ROWS, _CH), jnp.float32),
            pltpu.VMEM((D * TS, _CH), jnp.float32),
            pltpu.VMEM((8, _CH), jnp.int32),
            pltpu.VMEM((16,), jnp.float32),
            pltpu.SemaphoreType.DMA((2,)),
        ],
        compiler_params=pltpu.CompilerParams(needs_layout_passes=False),
    )


def kernel(poses_reg, poses_cls, targets, plan_anchor):
    pa_t = jnp.transpose(plan_anchor, (1, 2, 3, 0)).reshape(NM * TS, 2, BS)
    pr_t = jnp.transpose(poses_reg, (1, 3, 2, 0)).reshape(ROWS, BS)
    tg_t = jnp.transpose(targets, (2, 1, 0)).reshape(D * TS, BS)
    cls_t = jnp.transpose(poses_cls, (1, 0))
    m_c = jnp.asarray(_M_NP)

    mi8, accc = pl.pallas_call(
        _k1_body,
        grid=(BS // B1,),
        in_specs=[
            pl.BlockSpec((NM * TS, 2, B1), lambda i: (0, 0, i)),
            pl.BlockSpec((D * TS, B1), lambda i: (0, i)),
            pl.BlockSpec((NM, B1), lambda i: (0, i)),
            pl.BlockSpec((NM, NM * 16), lambda i: (0, 0)),
        ],
        out_specs=[
            pl.BlockSpec((8, B1), lambda i: (0, i)),
            pl.BlockSpec((1, 1), lambda i: (0, 0)),
        ],
        out_shape=[
            jax.ShapeDtypeStruct((8, BS), jnp.int32),
            jax.ShapeDtypeStruct((1, 1), jnp.float32),
        ],
    )(pa_t, tg_t, cls_t, m_c)

    l1p = _make_k2()(pr_t, tg_t, mi8)

    if SC_LANES < BS:
        accl = pl.pallas_call(
            _k3b_body,
            grid=((BS - SC_LANES) // B1,),
            in_specs=[
                pl.BlockSpec((ROWS, B1), lambda i: (0, i + TC_BLK0)),
                pl.BlockSpec((D * TS, B1), lambda i: (0, i + TC_BLK0)),
                pl.BlockSpec((8, B1), lambda i: (0, i + TC_BLK0)),
            ],
            out_specs=pl.BlockSpec((1, 1), lambda i: (0, 0)),
            out_shape=jax.ShapeDtypeStruct((1, 1), jnp.float32),
        )(pr_t, tg_t, mi8)
        l1_tc = accl[0, 0]
    else:
        l1_tc = jnp.float32(0.0)

    return (accc[0, 0] * (10.0 / (BS * NM))
            + (l1_tc + jnp.sum(l1p)) * (1.0 / (BS * TS * D)))


# speedup vs baseline: 1.0252x; 1.0217x over previous
import numpy as np

import jax
import jax.numpy as jnp
from jax import lax
from jax.experimental import pallas as pl
from jax.experimental.pallas import tpu as pltpu
from jax.experimental.pallas import tpu_sc as plsc

BS, NM, TS, D = 16384, 20, 8, 3
ROWS = NM * D * TS
B1 = 2048
_NC, _NS = 2, 16
_NW = _NC * _NS
_CH = 128
SC_LANES = 12288
_NCH = SC_LANES // (_NW * _CH)
TC_BLK0 = SC_LANES // B1


def _pair_mask():
    m = np.zeros((NM, NM * 16), np.float32)
    for mm in range(NM):
        for t in range(TS):
            m[mm, mm * 16 + 2 * t] = 1.0
    return m


_M_NP = _pair_mask()


def _k1_body(pa_ref, tg_ref, m_ref, mi_ref):
    b = pa_ref.shape[2]
    tx = tg_ref[pl.ds(0, TS), :]
    ty = tg_ref[pl.ds(TS, TS), :]
    tgi = jnp.concatenate(
        [tx[:, None, :], ty[:, None, :]], axis=1).reshape(16, b)
    pa = pa_ref[...].reshape(NM, 16, b)
    d = pa - tgi[None]
    d2 = (d * d).reshape(NM * 16, b)
    sh = jnp.concatenate([d2[1:], d2[:1]], axis=0)
    h = jnp.sqrt(d2 + sh)
    dist = jnp.dot(m_ref[...], h, preferred_element_type=jnp.float32)
    mn = jnp.min(dist, axis=0, keepdims=True)
    i20 = lax.broadcasted_iota(jnp.int32, dist.shape, 0)
    cand = jnp.where(dist == mn, i20, jnp.int32(1 << 30))
    mi = jnp.min(cand, axis=0, keepdims=True)
    mi_ref[...] = jnp.broadcast_to(mi, (8, b))


def _k3a_body(cls_ref, mi_ref, accc_ref):
    @pl.when(pl.program_id(0) == 0)
    def _():
        accc_ref[...] = jnp.zeros_like(accc_ref)

    x = cls_ref[...]
    mi = mi_ref[pl.ds(0, 1), :]
    i20 = lax.broadcasted_iota(jnp.int32, x.shape, 0)
    t = (i20 == mi).astype(jnp.float32)
    p = jax.nn.sigmoid(x)
    pt = (1.0 - p) * t + p * (1.0 - t)
    w = (0.25 * t + 0.75 * (1.0 - t)) * pt * pt
    bce = jnp.maximum(x, 0.0) - x * t + jnp.log1p(jnp.exp(-jnp.abs(x)))
    accc_ref[...] += jnp.sum(bce * w).reshape(1, 1)


def _k3b_body(pr_ref, tg_ref, mi_ref, accl_ref):
    @pl.when(pl.program_id(0) == 0)
    def _():
        accl_ref[...] = jnp.zeros_like(accl_ref)

    tg = tg_ref[...]
    mi = mi_ref[pl.ds(0, 1), :]
    acc = jnp.zeros(mi.shape, jnp.float32)
    for m in range(NM):
        d = jnp.abs(pr_ref[pl.ds(m * 24, 24), :] - tg)
        s = jnp.sum(d, axis=0, keepdims=True)
        acc = acc + jnp.where(mi == m, s, 0.0)
    accl_ref[...] += jnp.sum(acc).reshape(1, 1)


def _k2_body(pr_hbm, tg_hbm, mi_hbm, out_hbm, slabs, tgv, miv, accv, sems):
    wid = lax.axis_index("s") * _NC + lax.axis_index("c")
    col16 = lax.iota(jnp.int32, 16)
    base = wid * _NCH * _CH

    def start_slab(chunk, buf):
        c0 = base + chunk * _CH
        return pltpu.async_copy(
            pr_hbm.at[:, pl.ds(c0, _CH)], slabs.at[buf], sems.at[buf])

    acc = jnp.zeros((16,), jnp.float32)
    pend = start_slab(0, 0)
    for chunk in range(_NCH):
        buf = chunk % 2
        if chunk + 1 < _NCH:
            nxt = start_slab(chunk + 1, 1 - buf)
        c0 = base + chunk * _CH
        pltpu.sync_copy(tg_hbm.at[:, pl.ds(c0, _CH)], tgv)
        pltpu.sync_copy(mi_hbm.at[:, pl.ds(c0, _CH)], miv)
        pend.wait()
        if chunk + 1 < _NCH:
            pend = nxt

        def body(g, s):
            mi16 = miv[0, pl.ds(g * 16, 16)]
            rowb = mi16 * (TS * D)
            cols = col16 + g * 16
            for r in range(TS * D):
                v = plsc.load_gather(slabs.at[buf], [rowb + r, cols])
                s = s + jnp.abs(v - tgv[r, pl.ds(g * 16, 16)])
            return s

        acc = lax.fori_loop(0, _CH // 16, body, acc)
    accv[...] = acc
    pltpu.sync_copy(accv, out_hbm.at[wid])


def _make_k2():
    mesh = plsc.VectorSubcoreMesh(core_axis_name="c", subcore_axis_name="s")
    return pl.kernel(
        _k2_body,
        out_type=jax.ShapeDtypeStruct((_NW, 16), jnp.float32),
        mesh=mesh,
        scratch_types=[
            pltpu.VMEM((2, ROWS, _CH), jnp.float32),
            pltpu.VMEM((D * TS, _CH), jnp.float32),
            pltpu.VMEM((8, _CH), jnp.int32),
            pltpu.VMEM((16,), jnp.float32),
            pltpu.SemaphoreType.DMA((2,)),
        ],
        compiler_params=pltpu.CompilerParams(needs_layout_passes=False),
    )


def kernel(poses_reg, poses_cls, targets, plan_anchor):
    pa_t = jnp.transpose(plan_anchor, (1, 2, 3, 0)).reshape(NM * TS, 2, BS)
    pr_t = jnp.transpose(poses_reg, (1, 3, 2, 0)).reshape(ROWS, BS)
    tg_t = jnp.transpose(targets, (2, 1, 0)).reshape(D * TS, BS)
    cls_t = jnp.transpose(poses_cls, (1, 0))
    m_c = jnp.asarray(_M_NP)

    mi8 = pl.pallas_call(
        _k1_body,
        grid=(BS // B1,),
        in_specs=[
            pl.BlockSpec((NM * TS, 2, B1), lambda i: (0, 0, i)),
            pl.BlockSpec((D * TS, B1), lambda i: (0, i)),
            pl.BlockSpec((NM, NM * 16), lambda i: (0, 0)),
        ],
        out_specs=pl.BlockSpec((8, B1), lambda i: (0, i)),
        out_shape=jax.ShapeDtypeStruct((8, BS), jnp.int32),
    )(pa_t, tg_t, m_c)

    l1p = _make_k2()(pr_t, tg_t, mi8)

    accc = pl.pallas_call(
        _k3a_body,
        grid=(BS // B1,),
        in_specs=[
            pl.BlockSpec((NM, B1), lambda i: (0, i)),
            pl.BlockSpec((8, B1), lambda i: (0, i)),
        ],
        out_specs=pl.BlockSpec((1, 1), lambda i: (0, 0)),
        out_shape=jax.ShapeDtypeStruct((1, 1), jnp.float32),
    )(cls_t, mi8)

    if SC_LANES < BS:
        accl = pl.pallas_call(
            _k3b_body,
            grid=((BS - SC_LANES) // B1,),
            in_specs=[
                pl.BlockSpec((ROWS, B1), lambda i: (0, i + TC_BLK0)),
                pl.BlockSpec((D * TS, B1), lambda i: (0, i + TC_BLK0)),
                pl.BlockSpec((8, B1), lambda i: (0, i + TC_BLK0)),
            ],
            out_specs=pl.BlockSpec((1, 1), lambda i: (0, 0)),
            out_shape=jax.ShapeDtypeStruct((1, 1), jnp.float32),
        )(pr_t, tg_t, mi8)
        l1_tc = accl[0, 0]
    else:
        l1_tc = jnp.float32(0.0)

    return (accc[0, 0] * (10.0 / (BS * NM))
            + (l1_tc + jnp.sum(l1p)) * (1.0 / (BS * TS * D)))


# speedup vs baseline: 1.1005x; 1.0735x over previous
import numpy as np

import jax
import jax.numpy as jnp
from jax import lax
from jax.experimental import pallas as pl
from jax.experimental.pallas import tpu as pltpu
from jax.experimental.pallas import tpu_sc as plsc

BS, NM, TS, D = 16384, 20, 8, 3
ROWS = NM * D * TS
B1 = 2048
_NC, _NS = 2, 16
_NW = _NC * _NS
_CH = 128
SC_LANES = 12288
_NCH = SC_LANES // (_NW * _CH)
TC_BLK0 = SC_LANES // B1


def _pair_mask():
    m = np.zeros((NM, NM * 16), np.float32)
    for mm in range(NM):
        for t in range(TS):
            m[mm, mm * 16 + 2 * t] = 1.0
    return m


_M_NP = _pair_mask()


def _k1_body(pa_ref, tg_ref, mi_ref):
    b = pa_ref.shape[2]
    tx = tg_ref[pl.ds(0, TS), :]
    ty = tg_ref[pl.ds(TS, TS), :]
    txb = jnp.broadcast_to(tx[None], (NM, TS, b)).reshape(NM * TS, b)
    tyb = jnp.broadcast_to(ty[None], (NM, TS, b)).reshape(NM * TS, b)
    dx = pa_ref[:, 0, :] - txb
    dy = pa_ref[:, 1, :] - tyb
    h = jnp.sqrt(dx * dx + dy * dy)
    dist = jnp.sum(h.reshape(NM, TS, b), axis=1)
    mn = jnp.min(dist, axis=0, keepdims=True)
    i20 = lax.broadcasted_iota(jnp.int32, dist.shape, 0)
    cand = jnp.where(dist == mn, i20, jnp.int32(1 << 30))
    mi = jnp.min(cand, axis=0, keepdims=True)
    mi_ref[...] = jnp.broadcast_to(mi, (8, b))


def _k3a_body(cls_ref, mi_ref, accc_ref):
    @pl.when(pl.program_id(0) == 0)
    def _():
        accc_ref[...] = jnp.zeros_like(accc_ref)

    x = cls_ref[...]
    mi = mi_ref[pl.ds(0, 1), :]
    i20 = lax.broadcasted_iota(jnp.int32, x.shape, 0)
    t = (i20 == mi).astype(jnp.float32)
    p = jax.nn.sigmoid(x)
    pt = (1.0 - p) * t + p * (1.0 - t)
    w = (0.25 * t + 0.75 * (1.0 - t)) * pt * pt
    bce = jnp.maximum(x, 0.0) - x * t + jnp.log1p(jnp.exp(-jnp.abs(x)))
    accc_ref[...] += jnp.sum(bce * w).reshape(1, 1)


def _k3b_body(pr_ref, tg_ref, mi_ref, accl_ref):
    @pl.when(pl.program_id(0) == 0)
    def _():
        accl_ref[...] = jnp.zeros_like(accl_ref)

    tg = tg_ref[...]
    mi = mi_ref[pl.ds(0, 1), :]
    acc = jnp.zeros(mi.shape, jnp.float32)
    for m in range(NM):
        d = jnp.abs(pr_ref[pl.ds(m * 24, 24), :] - tg)
        s = jnp.sum(d, axis=0, keepdims=True)
        acc = acc + jnp.where(mi == m, s, 0.0)
    accl_ref[...] += jnp.sum(acc).reshape(1, 1)


def _k2_body(pr_hbm, tg_hbm, mi_hbm, out_hbm, slabs, tgv, miv, accv, sems):
    wid = lax.axis_index("s") * _NC + lax.axis_index("c")
    col16 = lax.iota(jnp.int32, 16)
    base = wid * _NCH * _CH

    def start_chunk(chunk, buf):
        c0 = base + chunk * _CH
        return (
            pltpu.async_copy(
                pr_hbm.at[:, pl.ds(c0, _CH)], slabs.at[buf], sems.at[buf]),
            pltpu.async_copy(
                tg_hbm.at[:, pl.ds(c0, _CH)], tgv.at[buf], sems.at[buf]),
            pltpu.async_copy(
                mi_hbm.at[pl.ds(0, 1), pl.ds(c0, _CH)], miv.at[buf],
                sems.at[buf]),
        )

    acc = jnp.zeros((16,), jnp.float32)
    pend = start_chunk(0, 0)
    for chunk in range(_NCH):
        buf = chunk % 2
        if chunk + 1 < _NCH:
            nxt = start_chunk(chunk + 1, 1 - buf)
        for c in pend:
            c.wait()
        if chunk + 1 < _NCH:
            pend = nxt

        def body(g, s):
            mi16 = miv[buf, 0, pl.ds(g * 16, 16)]
            rowb = mi16 * (TS * D)
            cols = col16 + g * 16
            for r in range(TS * D):
                v = plsc.load_gather(slabs.at[buf], [rowb + r, cols])
                s = s + jnp.abs(v - tgv[buf, r, pl.ds(g * 16, 16)])
            return s

        acc = lax.fori_loop(0, _CH // 16, body, acc)
    accv[...] = acc
    pltpu.sync_copy(accv, out_hbm.at[wid])


def _make_k2():
    mesh = plsc.VectorSubcoreMesh(core_axis_name="c", subcore_axis_name="s")
    return pl.kernel(
        _k2_body,
        out_type=jax.ShapeDtypeStruct((_NW, 16), jnp.float32),
        mesh=mesh,
        scratch_types=[
            pltpu.VMEM((2, ROWS, _CH), jnp.float32),
            pltpu.VMEM((2, D * TS, _CH), jnp.float32),
            pltpu.VMEM((2, 1, _CH), jnp.int32),
            pltpu.VMEM((16,), jnp.float32),
            pltpu.SemaphoreType.DMA((2,)),
        ],
        compiler_params=pltpu.CompilerParams(needs_layout_passes=False),
    )


def kernel(poses_reg, poses_cls, targets, plan_anchor):
    pa_t = jnp.transpose(plan_anchor, (1, 2, 3, 0)).reshape(NM * TS, 2, BS)
    pr_t = jnp.transpose(poses_reg, (1, 3, 2, 0)).reshape(ROWS, BS)
    tg_t = jnp.transpose(targets, (2, 1, 0)).reshape(D * TS, BS)
    cls_t = jnp.transpose(poses_cls, (1, 0))
    m_c = jnp.asarray(_M_NP)

    mi8 = pl.pallas_call(
        _k1_body,
        grid=(BS // B1,),
        in_specs=[
            pl.BlockSpec((NM * TS, 2, B1), lambda i: (0, 0, i)),
            pl.BlockSpec((D * TS, B1), lambda i: (0, i)),
        ],
        out_specs=pl.BlockSpec((8, B1), lambda i: (0, i)),
        out_shape=jax.ShapeDtypeStruct((8, BS), jnp.int32),
    )(pa_t, tg_t)

    l1p = _make_k2()(pr_t, tg_t, mi8)

    accc = pl.pallas_call(
        _k3a_body,
        grid=(BS // B1,),
        in_specs=[
            pl.BlockSpec((NM, B1), lambda i: (0, i)),
            pl.BlockSpec((8, B1), lambda i: (0, i)),
        ],
        out_specs=pl.BlockSpec((1, 1), lambda i: (0, 0)),
        out_shape=jax.ShapeDtypeStruct((1, 1), jnp.float32),
    )(cls_t, mi8)

    if SC_LANES < BS:
        accl = pl.pallas_call(
            _k3b_body,
            grid=((BS - SC_LANES) // B1,),
            in_specs=[
                pl.BlockSpec((ROWS, B1), lambda i: (0, i + TC_BLK0)),
                pl.BlockSpec((D * TS, B1), lambda i: (0, i + TC_BLK0)),
                pl.BlockSpec((8, B1), lambda i: (0, i + TC_BLK0)),
            ],
            out_specs=pl.BlockSpec((1, 1), lambda i: (0, 0)),
            out_shape=jax.ShapeDtypeStruct((1, 1), jnp.float32),
        )(pr_t, tg_t, mi8)
        l1_tc = accl[0, 0]
    else:
        l1_tc = jnp.float32(0.0)

    return (accc[0, 0] * (10.0 / (BS * NM))
            + (l1_tc + jnp.sum(l1p)) * (1.0 / (BS * TS * D)))
